# Initial kernel scaffold; baseline (speedup 1.0000x reference)
import jax
import jax.numpy as jnp
from jax.experimental import pallas as pl


def kernel(x, gate_W, w1, w2):
    raise NotImplementedError("write your pallas kernel here")


# speedup vs baseline: 4.7404x; 4.7404x over previous
import functools
import numpy as np
import jax
import jax.numpy as jnp
from jax import lax
from jax.experimental import pallas as pl
from jax.experimental.pallas import tpu as pltpu
from jax.experimental.pallas import tpu_sc as plsc

L_TOK = 2048
D = 1024
F = 4096
E = 16
RB = 256
NB = 32
NSLOT = NB * RB
NW = 32

_SQRT1_2 = 0.7071067811865476


def _routing_body(x_ref, gwt_ref, pos0_ref, pos1_ref, w0_ref, w1_ref,
                  be_ref, valid_ref):
    x = x_ref[...]
    gwt = gwt_ref[...]
    logits = jnp.dot(x, gwt, preferred_element_type=jnp.float32)

    eio = lax.broadcasted_iota(jnp.int32, (L_TOK, E), 1).astype(jnp.float32)
    m1 = jnp.max(logits, axis=1, keepdims=True)
    a1 = jnp.min(jnp.where(logits == m1, eio, float(E)), axis=1, keepdims=True)
    oh1 = (eio == a1).astype(jnp.float32)
    masked = jnp.where(oh1 > 0, -jnp.inf, logits)
    m2 = jnp.max(masked, axis=1, keepdims=True)
    a2 = jnp.min(jnp.where(masked == m2, eio, float(E)), axis=1, keepdims=True)
    oh2 = (eio == a2).astype(jnp.float32)

    p1 = 1.0 / (1.0 + jnp.exp(m2 - m1))
    p2 = 1.0 - p1
    ones_row = jnp.ones((1, E), jnp.float32)
    w0_ref[...] = p1 * ones_row
    w1_ref[...] = p2 * ones_row

    ot = oh1 + oh2
    rio = lax.broadcasted_iota(jnp.int32, (128, 128), 0).astype(jnp.float32)
    cio = lax.broadcasted_iota(jnp.int32, (128, 128), 1).astype(jnp.float32)
    tri128 = (rio > cio).astype(jnp.float32)
    blocks = []
    for b in range(L_TOK // 128):
        ob = lax.slice(ot, (b * 128, 0), ((b + 1) * 128, E))
        blocks.append(jnp.dot(tri128, ob, preferred_element_type=jnp.float32))
    ct = jnp.concatenate(blocks, axis=0)
    bsum = jnp.concatenate(
        [jnp.sum(lax.slice(ot, (b * 128, 0), ((b + 1) * 128, E)),
                 axis=0, keepdims=True) for b in range(L_TOK // 128)], axis=0)
    rio16 = lax.broadcasted_iota(jnp.int32, (16, 16), 0).astype(jnp.float32)
    cio16 = lax.broadcasted_iota(jnp.int32, (16, 16), 1).astype(jnp.float32)
    tri16 = (rio16 > cio16).astype(jnp.float32)
    boffs = jnp.dot(tri16, bsum, preferred_element_type=jnp.float32)
    tid = lax.broadcasted_iota(jnp.int32, (L_TOK, 1), 0).astype(jnp.float32)
    blk_of_t = jnp.floor(tid / 128.0)
    sel = (lax.broadcasted_iota(jnp.int32, (L_TOK, 16), 1).astype(jnp.float32) ==
           blk_of_t).astype(jnp.float32)
    ct = ct + jnp.dot(sel, boffs, preferred_element_type=jnp.float32)

    cnt = jnp.sum(ot, axis=0, keepdims=True)
    pblk = jnp.floor((cnt + float(RB - 1)) / float(RB))
    pad = pblk * float(RB)
    trU = (rio16 < cio16).astype(jnp.float32)
    start = jnp.dot(pad, trU, preferred_element_type=jnp.float32)

    rank0 = jnp.sum(ct * oh1, axis=1, keepdims=True)
    rank1 = jnp.sum(ct * oh2, axis=1, keepdims=True)
    s0 = jnp.sum(start * oh1, axis=1, keepdims=True)
    s1 = jnp.sum(start * oh2, axis=1, keepdims=True)
    pos0_ref[...] = (s0 + rank0).astype(jnp.int32)
    pos1_ref[...] = (s1 + rank1).astype(jnp.int32)

    sblk = start / float(RB)
    bio = lax.broadcasted_iota(jnp.int32, (NB, 1), 0).astype(jnp.float32)
    ind = jnp.logical_and(bio >= sblk, bio < sblk + pblk)
    indf = ind.astype(jnp.float32)
    eio_r = lax.broadcasted_iota(jnp.int32, (1, E), 1).astype(jnp.float32)
    bev = jnp.sum(indf * eio_r, axis=1, keepdims=True)
    vv = jnp.sum(indf, axis=1, keepdims=True)
    max_used = jnp.max(jnp.where(cnt > 0, eio_r, -1.0))
    be_ref[...] = jnp.where(vv > 0, bev, max_used).astype(jnp.int32)
    valid_ref[...] = vv.astype(jnp.int32)


def _routing(x_flat, gwt):
    outs = [
        jax.ShapeDtypeStruct((L_TOK, 1), jnp.int32),
        jax.ShapeDtypeStruct((L_TOK, 1), jnp.int32),
        jax.ShapeDtypeStruct((L_TOK, E), jnp.float32),
        jax.ShapeDtypeStruct((L_TOK, E), jnp.float32),
        jax.ShapeDtypeStruct((NB, 1), jnp.int32),
        jax.ShapeDtypeStruct((NB, 1), jnp.int32),
    ]
    return pl.pallas_call(_routing_body, out_shape=outs)(x_flat, gwt)


FT = 2048
NF = F // FT


def _ffn_body(be_ref, valid_ref, xs_ref, w1_ref, w2_ref, y_ref):
    b = pl.program_id(0)
    f = pl.program_id(1)

    @pl.when(valid_ref[b] != 0)
    def _():
        xb = xs_ref[...]
        h = jnp.dot(xb, w1_ref[0], preferred_element_type=jnp.float32)
        h = 0.5 * h * (1.0 + lax.erf(h * _SQRT1_2))
        part = jnp.dot(h, w2_ref[0], preferred_element_type=jnp.float32)

        @pl.when(f == 0)
        def _():
            y_ref[...] = part

        @pl.when(f != 0)
        def _():
            y_ref[...] += part


def _ffn(be, valid, xs, w1, w2):
    grid_spec = pltpu.PrefetchScalarGridSpec(
        num_scalar_prefetch=2,
        grid=(NB, NF),
        in_specs=[
            pl.BlockSpec((RB, D), lambda b, f, be, v: (b, 0)),
            pl.BlockSpec((1, D, FT), lambda b, f, be, v: (be[b], 0, f)),
            pl.BlockSpec((1, FT, D), lambda b, f, be, v: (be[b], f, 0)),
        ],
        out_specs=pl.BlockSpec((RB, D), lambda b, f, be, v: (b, 0)),
    )
    return pl.pallas_call(
        _ffn_body,
        grid_spec=grid_spec,
        out_shape=jax.ShapeDtypeStruct((NSLOT, D), jnp.float32),
        compiler_params=pltpu.CompilerParams(
            dimension_semantics=("arbitrary", "arbitrary")),
    )(be, valid, xs, w1, w2)


def _dispatch_body(x_hbm, tok_hbm, pos_hbm, xs_hbm, tokv, posv, rows, sem):
    wid = lax.axis_index("s") * 2 + lax.axis_index("c")
    for h in range(2):
        r = wid * 2 + h
        pltpu.sync_copy(tok_hbm.at[r], tokv)
        pltpu.sync_copy(pos_hbm.at[r], posv)
        pltpu.async_copy(x_hbm.at[tokv], rows, sem).wait()
        pltpu.async_copy(rows, xs_hbm.at[posv], sem).wait()


def _dispatch(x_flat, tok, pos):
    mesh = plsc.VectorSubcoreMesh(core_axis_name="c", subcore_axis_name="s")
    kfn = functools.partial(
        pl.kernel, mesh=mesh,
        out_type=jax.ShapeDtypeStruct((NSLOT, D), jnp.float32),
        scratch_types=[
            pltpu.VMEM((64,), jnp.int32),
            pltpu.VMEM((64,), jnp.int32),
            pltpu.VMEM((64, D), jnp.float32),
            pltpu.SemaphoreType.DMA,
        ],
    )(_dispatch_body)
    return kfn(x_flat, tok, pos)


def _combine_body(y_hbm, pos0_hbm, pos1_hbm, w0_hbm, w1_hbm, out_hbm,
                  p0v, p1v, buf0, buf1, wv0, wv1, sem):
    wid = lax.axis_index("s") * 2 + lax.axis_index("c")
    base = wid * 64
    pltpu.sync_copy(w0_hbm.at[pl.ds(base * E, 64 * E)], wv0)
    pltpu.sync_copy(w1_hbm.at[pl.ds(base * E, 64 * E)], wv1)
    for h in range(2):
        b2 = base + h * 32
        pltpu.sync_copy(pos0_hbm.at[pl.ds(b2, 32)], p0v)
        pltpu.sync_copy(pos1_hbm.at[pl.ds(b2, 32)], p1v)
        pltpu.async_copy(y_hbm.at[p0v], buf0, sem).wait()
        pltpu.async_copy(y_hbm.at[p1v], buf1, sem).wait()

        def body(j, _):
            jj = h * 32 + j
            a = wv0[pl.ds(jj * E, E)]
            bwt = wv1[pl.ds(jj * E, E)]
            for c in range(D // E):
                s = c * E
                v = (buf0[j, pl.ds(s, E)] * a + buf1[j, pl.ds(s, E)] * bwt)
                buf0[j, pl.ds(s, E)] = v
            return 0

        lax.fori_loop(0, 32, body, 0)
        pltpu.sync_copy(buf0, out_hbm.at[pl.ds(b2, 32)])


def _combine(y, pos0, pos1, w0f, w1f):
    mesh = plsc.VectorSubcoreMesh(core_axis_name="c", subcore_axis_name="s")
    kfn = functools.partial(
        pl.kernel, mesh=mesh,
        out_type=jax.ShapeDtypeStruct((L_TOK, D), jnp.float32),
        scratch_types=[
            pltpu.VMEM((32,), jnp.int32),
            pltpu.VMEM((32,), jnp.int32),
            pltpu.VMEM((32, D), jnp.float32),
            pltpu.VMEM((32, D), jnp.float32),
            pltpu.VMEM((64 * E,), jnp.float32),
            pltpu.VMEM((64 * E,), jnp.float32),
            pltpu.SemaphoreType.DMA,
        ],
    )(_combine_body)
    return kfn(y, pos0, pos1, w0f, w1f)


_TOK_IDS = np.arange(2 * L_TOK, dtype=np.int32) // 2


def kernel(x, gate_W, w1, w2):
    Bq, Lq, Dq = x.shape
    x_flat = x.reshape(-1, Dq)
    pos0, pos1, w0s, w1s, be, valid = _routing(x_flat, gate_W.T)

    pos0 = pos0.reshape(-1)
    pos1 = pos1.reshape(-1)
    posd = jnp.stack([pos0, pos1], axis=1).reshape(NW * 2, 64)
    tok = jnp.asarray(_TOK_IDS).reshape(NW * 2, 64)

    xs = _dispatch(x_flat, tok, posd)
    y = _ffn(be.reshape(-1), valid.reshape(-1), xs, w1, w2)
    out = _combine(y, pos0, pos1, w0s.reshape(-1), w1s.reshape(-1))
    return out.reshape(Bq, Lq, Dq)


# speedup vs baseline: 5.6987x; 1.2022x over previous
import functools
import numpy as np
import jax
import jax.numpy as jnp
from jax import lax
from jax.experimental import pallas as pl
from jax.experimental.pallas import tpu as pltpu
from jax.experimental.pallas import tpu_sc as plsc

L_TOK = 2048
D = 1024
F = 4096
E = 16
RB = 512
NB = 23
NBPAD = 32
NSLOT = NB * RB
NW = 32

_SQRT1_2 = 0.7071067811865476


def _routing_body(x_ref, gwt_ref, pos0_ref, pos1_ref, w0_ref, w1_ref,
                  be_ref, valid_ref):
    x = x_ref[...]
    gwt = gwt_ref[...]
    logits = jnp.dot(x, gwt, preferred_element_type=jnp.float32)

    eio = lax.broadcasted_iota(jnp.int32, (L_TOK, E), 1).astype(jnp.float32)
    m1 = jnp.max(logits, axis=1, keepdims=True)
    a1 = jnp.min(jnp.where(logits == m1, eio, float(E)), axis=1, keepdims=True)
    oh1 = (eio == a1).astype(jnp.float32)
    masked = jnp.where(oh1 > 0, -jnp.inf, logits)
    m2 = jnp.max(masked, axis=1, keepdims=True)
    a2 = jnp.min(jnp.where(masked == m2, eio, float(E)), axis=1, keepdims=True)
    oh2 = (eio == a2).astype(jnp.float32)

    p1 = 1.0 / (1.0 + jnp.exp(m2 - m1))
    p2 = 1.0 - p1
    ones_row = jnp.ones((1, E), jnp.float32)
    w0_ref[...] = p1 * ones_row
    w1_ref[...] = p2 * ones_row

    ot = oh1 + oh2
    rio = lax.broadcasted_iota(jnp.int32, (128, 128), 0).astype(jnp.float32)
    cio = lax.broadcasted_iota(jnp.int32, (128, 128), 1).astype(jnp.float32)
    tri128 = (rio > cio).astype(jnp.float32)
    blocks = []
    for b in range(L_TOK // 128):
        ob = lax.slice(ot, (b * 128, 0), ((b + 1) * 128, E))
        blocks.append(jnp.dot(tri128, ob, preferred_element_type=jnp.float32))
    ct = jnp.concatenate(blocks, axis=0)
    bsum = jnp.concatenate(
        [jnp.sum(lax.slice(ot, (b * 128, 0), ((b + 1) * 128, E)),
                 axis=0, keepdims=True) for b in range(L_TOK // 128)], axis=0)
    rio16 = lax.broadcasted_iota(jnp.int32, (16, 16), 0).astype(jnp.float32)
    cio16 = lax.broadcasted_iota(jnp.int32, (16, 16), 1).astype(jnp.float32)
    tri16 = (rio16 > cio16).astype(jnp.float32)
    boffs = jnp.dot(tri16, bsum, preferred_element_type=jnp.float32)
    tid = lax.broadcasted_iota(jnp.int32, (L_TOK, 1), 0).astype(jnp.float32)
    blk_of_t = jnp.floor(tid / 128.0)
    sel = (lax.broadcasted_iota(jnp.int32, (L_TOK, 16), 1).astype(jnp.float32) ==
           blk_of_t).astype(jnp.float32)
    ct = ct + jnp.dot(sel, boffs, preferred_element_type=jnp.float32)

    cnt = jnp.sum(ot, axis=0, keepdims=True)
    pblk = jnp.floor((cnt + float(RB - 1)) / float(RB))
    pad = pblk * float(RB)
    trU = (rio16 < cio16).astype(jnp.float32)
    start = jnp.dot(pad, trU, preferred_element_type=jnp.float32)

    rank0 = jnp.sum(ct * oh1, axis=1, keepdims=True)
    rank1 = jnp.sum(ct * oh2, axis=1, keepdims=True)
    s0 = jnp.sum(start * oh1, axis=1, keepdims=True)
    s1 = jnp.sum(start * oh2, axis=1, keepdims=True)
    pos0_ref[...] = (s0 + rank0).astype(jnp.int32)
    pos1_ref[...] = (s1 + rank1).astype(jnp.int32)

    sblk = start / float(RB)
    bio = lax.broadcasted_iota(jnp.int32, (NBPAD, 1), 0).astype(jnp.float32)
    ind = jnp.logical_and(bio >= sblk, bio < sblk + pblk)
    indf = ind.astype(jnp.float32)
    eio_r = lax.broadcasted_iota(jnp.int32, (1, E), 1).astype(jnp.float32)
    bev = jnp.sum(indf * eio_r, axis=1, keepdims=True)
    vv = jnp.sum(indf, axis=1, keepdims=True)
    max_used = jnp.max(jnp.where(cnt > 0, eio_r, -1.0))
    be_ref[...] = jnp.where(vv > 0, bev, max_used).astype(jnp.int32)
    valid_ref[...] = vv.astype(jnp.int32)


def _routing(x_flat, gwt):
    outs = [
        jax.ShapeDtypeStruct((L_TOK, 1), jnp.int32),
        jax.ShapeDtypeStruct((L_TOK, 1), jnp.int32),
        jax.ShapeDtypeStruct((L_TOK, E), jnp.float32),
        jax.ShapeDtypeStruct((L_TOK, E), jnp.float32),
        jax.ShapeDtypeStruct((NBPAD, 1), jnp.int32),
        jax.ShapeDtypeStruct((NBPAD, 1), jnp.int32),
    ]
    return pl.pallas_call(_routing_body, out_shape=outs)(x_flat, gwt)


FT = 2048
NF = F // FT


def _ffn_body(be_ref, valid_ref, xs_ref, w1_ref, w2_ref, y_ref):
    b = pl.program_id(0)
    f = pl.program_id(1)

    @pl.when(valid_ref[b] != 0)
    def _():
        xb = xs_ref[...]
        h = jnp.dot(xb, w1_ref[0], preferred_element_type=jnp.float32)
        h = 0.5 * h * (1.0 + lax.erf(h * _SQRT1_2))
        part = jnp.dot(h, w2_ref[0], preferred_element_type=jnp.float32)

        @pl.when(f == 0)
        def _():
            y_ref[...] = part

        @pl.when(f != 0)
        def _():
            y_ref[...] += part


def _ffn(be, valid, xs, w1, w2):
    grid_spec = pltpu.PrefetchScalarGridSpec(
        num_scalar_prefetch=2,
        grid=(NB, NF),
        in_specs=[
            pl.BlockSpec((RB, D), lambda b, f, be, v: (b, 0)),
            pl.BlockSpec((1, D, FT), lambda b, f, be, v: (be[b], 0, f)),
            pl.BlockSpec((1, FT, D), lambda b, f, be, v: (be[b], f, 0)),
        ],
        out_specs=pl.BlockSpec((RB, D), lambda b, f, be, v: (b, 0)),
    )
    return pl.pallas_call(
        _ffn_body,
        grid_spec=grid_spec,
        out_shape=jax.ShapeDtypeStruct((NSLOT, D), jnp.float32),
        compiler_params=pltpu.CompilerParams(
            dimension_semantics=("arbitrary", "arbitrary")),
    )(be, valid, xs, w1, w2)


def _dispatch_body(x_hbm, tok_hbm, pos_hbm, xs_hbm, tokv, posv, rows, sem):
    wid = lax.axis_index("s") * 2 + lax.axis_index("c")
    for h in range(2):
        r = wid * 2 + h
        pltpu.sync_copy(tok_hbm.at[r], tokv)
        pltpu.sync_copy(pos_hbm.at[r], posv)
        pltpu.async_copy(x_hbm.at[tokv], rows, sem).wait()
        pltpu.async_copy(rows, xs_hbm.at[posv], sem).wait()


def _dispatch(x_flat, tok, pos):
    mesh = plsc.VectorSubcoreMesh(core_axis_name="c", subcore_axis_name="s")
    kfn = functools.partial(
        pl.kernel, mesh=mesh,
        out_type=jax.ShapeDtypeStruct((NSLOT, D), jnp.float32),
        scratch_types=[
            pltpu.VMEM((64,), jnp.int32),
            pltpu.VMEM((64,), jnp.int32),
            pltpu.VMEM((64, D), jnp.float32),
            pltpu.SemaphoreType.DMA,
        ],
    )(_dispatch_body)
    return kfn(x_flat, tok, pos)


def _combine_body(y_hbm, pos0_hbm, pos1_hbm, w0_hbm, w1_hbm, out_hbm,
                  p0v, p1v, buf0, buf1, wv0, wv1, sem):
    wid = lax.axis_index("s") * 2 + lax.axis_index("c")
    base = wid * 64
    pltpu.sync_copy(w0_hbm.at[pl.ds(base * E, 64 * E)], wv0)
    pltpu.sync_copy(w1_hbm.at[pl.ds(base * E, 64 * E)], wv1)
    for h in range(2):
        b2 = base + h * 32
        pltpu.sync_copy(pos0_hbm.at[pl.ds(b2, 32)], p0v)
        pltpu.sync_copy(pos1_hbm.at[pl.ds(b2, 32)], p1v)
        pltpu.async_copy(y_hbm.at[p0v], buf0, sem).wait()
        pltpu.async_copy(y_hbm.at[p1v], buf1, sem).wait()

        def body(j, _):
            jj = h * 32 + j
            a = wv0[pl.ds(jj * E, E)]
            bwt = wv1[pl.ds(jj * E, E)]
            for c in range(D // E):
                s = c * E
                v = (buf0[j, pl.ds(s, E)] * a + buf1[j, pl.ds(s, E)] * bwt)
                buf0[j, pl.ds(s, E)] = v
            return 0

        lax.fori_loop(0, 32, body, 0)
        pltpu.sync_copy(buf0, out_hbm.at[pl.ds(b2, 32)])


def _combine(y, pos0, pos1, w0f, w1f):
    mesh = plsc.VectorSubcoreMesh(core_axis_name="c", subcore_axis_name="s")
    kfn = functools.partial(
        pl.kernel, mesh=mesh,
        out_type=jax.ShapeDtypeStruct((L_TOK, D), jnp.float32),
        scratch_types=[
            pltpu.VMEM((32,), jnp.int32),
            pltpu.VMEM((32,), jnp.int32),
            pltpu.VMEM((32, D), jnp.float32),
            pltpu.VMEM((32, D), jnp.float32),
            pltpu.VMEM((64 * E,), jnp.float32),
            pltpu.VMEM((64 * E,), jnp.float32),
            pltpu.SemaphoreType.DMA,
        ],
    )(_combine_body)
    return kfn(y, pos0, pos1, w0f, w1f)


_TOK_IDS = np.arange(2 * L_TOK, dtype=np.int32) // 2


def kernel(x, gate_W, w1, w2):
    Bq, Lq, Dq = x.shape
    x_flat = x.reshape(-1, Dq)
    pos0, pos1, w0s, w1s, be, valid = _routing(x_flat, gate_W.T)

    pos0 = pos0.reshape(-1)
    pos1 = pos1.reshape(-1)
    posd = jnp.stack([pos0, pos1], axis=1).reshape(NW * 2, 64)
    tok = jnp.asarray(_TOK_IDS).reshape(NW * 2, 64)

    xs = _dispatch(x_flat, tok, posd)
    y = _ffn(be.reshape(-1), valid.reshape(-1), xs, w1, w2)
    out = _combine(y, pos0, pos1, w0s.reshape(-1), w1s.reshape(-1))
    return out.reshape(Bq, Lq, Dq)


# speedup vs baseline: 5.6995x; 1.0001x over previous
import functools
import numpy as np
import jax
import jax.numpy as jnp
from jax import lax
from jax.experimental import pallas as pl
from jax.experimental.pallas import tpu as pltpu
from jax.experimental.pallas import tpu_sc as plsc

L_TOK = 2048
D = 1024
F = 4096
E = 16
RB = 512
NB = 23
NBPAD = 32
NSTEP = 48
NSLOT = NB * RB
NW = 32

_SQRT1_2 = 0.7071067811865476


def _routing_body(logits_ref, pos0_ref, pos1_ref, w0_ref, w1_ref,
                  be_ref, valid_ref):
    logits = logits_ref[...]

    eio = lax.broadcasted_iota(jnp.int32, (L_TOK, E), 1).astype(jnp.float32)
    m1 = jnp.max(logits, axis=1, keepdims=True)
    a1 = jnp.min(jnp.where(logits == m1, eio, float(E)), axis=1, keepdims=True)
    oh1 = (eio == a1).astype(jnp.float32)
    masked = jnp.where(oh1 > 0, -jnp.inf, logits)
    m2 = jnp.max(masked, axis=1, keepdims=True)
    a2 = jnp.min(jnp.where(masked == m2, eio, float(E)), axis=1, keepdims=True)
    oh2 = (eio == a2).astype(jnp.float32)

    p1 = 1.0 / (1.0 + jnp.exp(m2 - m1))
    p2 = 1.0 - p1
    ones_row = jnp.ones((1, E), jnp.float32)
    w0_ref[...] = p1 * ones_row
    w1_ref[...] = p2 * ones_row

    ot = oh1 + oh2
    rio = lax.broadcasted_iota(jnp.int32, (128, 128), 0).astype(jnp.float32)
    cio = lax.broadcasted_iota(jnp.int32, (128, 128), 1).astype(jnp.float32)
    tri128 = (rio > cio).astype(jnp.float32)
    blocks = []
    for b in range(L_TOK // 128):
        ob = lax.slice(ot, (b * 128, 0), ((b + 1) * 128, E))
        blocks.append(jnp.dot(tri128, ob, preferred_element_type=jnp.float32,
                              precision=lax.Precision.HIGHEST))
    ct = jnp.concatenate(blocks, axis=0)
    bsum = jnp.concatenate(
        [jnp.sum(lax.slice(ot, (b * 128, 0), ((b + 1) * 128, E)),
                 axis=0, keepdims=True) for b in range(L_TOK // 128)], axis=0)
    rio16 = lax.broadcasted_iota(jnp.int32, (16, 16), 0).astype(jnp.float32)
    cio16 = lax.broadcasted_iota(jnp.int32, (16, 16), 1).astype(jnp.float32)
    tri16 = (rio16 > cio16).astype(jnp.float32)
    boffs = jnp.dot(tri16, bsum, preferred_element_type=jnp.float32,
                    precision=lax.Precision.HIGHEST)
    tid = lax.broadcasted_iota(jnp.int32, (L_TOK, 1), 0).astype(jnp.float32)
    blk_of_t = jnp.floor(tid / 128.0)
    sel = (lax.broadcasted_iota(jnp.int32, (L_TOK, 16), 1).astype(jnp.float32) ==
           blk_of_t).astype(jnp.float32)
    ct = ct + jnp.dot(sel, boffs, preferred_element_type=jnp.float32,
                      precision=lax.Precision.HIGHEST)

    cnt = jnp.sum(ot, axis=0, keepdims=True)
    pblk = jnp.floor((cnt + float(RB - 1)) / float(RB))
    pad = pblk * float(RB)
    trU = (rio16 < cio16).astype(jnp.float32)
    start = jnp.dot(pad, trU, preferred_element_type=jnp.float32,
                    precision=lax.Precision.HIGHEST)

    rank0 = jnp.sum(ct * oh1, axis=1, keepdims=True)
    rank1 = jnp.sum(ct * oh2, axis=1, keepdims=True)
    s0 = jnp.sum(start * oh1, axis=1, keepdims=True)
    s1 = jnp.sum(start * oh2, axis=1, keepdims=True)
    pos0_ref[...] = (s0 + rank0).astype(jnp.int32)
    pos1_ref[...] = (s1 + rank1).astype(jnp.int32)

    sblk = start / float(RB)
    bio = lax.broadcasted_iota(jnp.int32, (NBPAD, 1), 0).astype(jnp.float32)
    ind = jnp.logical_and(bio >= sblk, bio < sblk + pblk)
    indf = ind.astype(jnp.float32)
    eio_r = lax.broadcasted_iota(jnp.int32, (1, E), 1).astype(jnp.float32)
    bev = jnp.sum(indf * eio_r, axis=1, keepdims=True)
    vv = jnp.sum(indf, axis=1, keepdims=True)
    max_used = jnp.max(jnp.where(cnt > 0, eio_r, -1.0))
    be_ref[...] = jnp.where(vv > 0, bev, max_used).astype(jnp.int32)
    valid_ref[...] = vv.astype(jnp.int32)


def _routing(x_flat, gwt):
    outs = [
        jax.ShapeDtypeStruct((L_TOK, 1), jnp.int32),
        jax.ShapeDtypeStruct((L_TOK, 1), jnp.int32),
        jax.ShapeDtypeStruct((L_TOK, E), jnp.float32),
        jax.ShapeDtypeStruct((L_TOK, E), jnp.float32),
        jax.ShapeDtypeStruct((NBPAD, 1), jnp.int32),
        jax.ShapeDtypeStruct((NBPAD, 1), jnp.int32),
    ]
    return pl.pallas_call(_routing_body, out_shape=outs)(x_flat @ gwt)


FT = 2048
NF = F // FT


def _ffn_body(be_ref, valid_ref, xs_ref, w1_ref, w2_ref, y_ref):
    b = pl.program_id(0)
    f = pl.program_id(1)

    @pl.when(valid_ref[b] != 0)
    def _():
        xb = xs_ref[...]
        h = jnp.dot(xb, w1_ref[0], preferred_element_type=jnp.float32)
        h = 0.5 * h * (1.0 + lax.erf(h * _SQRT1_2))
        part = jnp.dot(h, w2_ref[0], preferred_element_type=jnp.float32)

        @pl.when(f == 0)
        def _():
            y_ref[...] = part

        @pl.when(f != 0)
        def _():
            y_ref[...] += part


def _ffn(be, valid, xs, w1, w2):
    grid_spec = pltpu.PrefetchScalarGridSpec(
        num_scalar_prefetch=2,
        grid=(NB, NF),
        in_specs=[
            pl.BlockSpec((RB, D), lambda b, f, be, v: (b, 0)),
            pl.BlockSpec((1, D, FT), lambda b, f, be, v: (be[b], 0, f)),
            pl.BlockSpec((1, FT, D), lambda b, f, be, v: (be[b], f, 0)),
        ],
        out_specs=pl.BlockSpec((RB, D), lambda b, f, be, v: (b, 0)),
    )
    return pl.pallas_call(
        _ffn_body,
        grid_spec=grid_spec,
        out_shape=jax.ShapeDtypeStruct((NSLOT, D), jnp.float32),
        compiler_params=pltpu.CompilerParams(
            dimension_semantics=("arbitrary", "arbitrary")),
    )(be, valid, xs, w1, w2)


def _dispatch_body(x_hbm, tok_hbm, pos_hbm, xs_hbm, tokv, posv, rows, sem):
    wid = lax.axis_index("s") * 2 + lax.axis_index("c")
    for h in range(2):
        r = wid * 2 + h
        pltpu.sync_copy(tok_hbm.at[r], tokv)
        pltpu.sync_copy(pos_hbm.at[r], posv)
        pltpu.async_copy(x_hbm.at[tokv], rows, sem).wait()
        pltpu.async_copy(rows, xs_hbm.at[posv], sem).wait()


def _dispatch(x_flat, tok, pos):
    mesh = plsc.VectorSubcoreMesh(core_axis_name="c", subcore_axis_name="s")
    kfn = functools.partial(
        pl.kernel, mesh=mesh,
        out_type=jax.ShapeDtypeStruct((NSLOT, D), jnp.float32),
        scratch_types=[
            pltpu.VMEM((64,), jnp.int32),
            pltpu.VMEM((64,), jnp.int32),
            pltpu.VMEM((64, D), jnp.float32),
            pltpu.SemaphoreType.DMA,
        ],
    )(_dispatch_body)
    return kfn(x_flat, tok, pos)


def _combine_body(y_hbm, pos0_hbm, pos1_hbm, w0_hbm, w1_hbm, out_hbm,
                  p0v, p1v, buf0, buf1, wv0, wv1, sem):
    wid = lax.axis_index("s") * 2 + lax.axis_index("c")
    base = wid * 64
    pltpu.sync_copy(w0_hbm.at[pl.ds(base * E, 64 * E)], wv0)
    pltpu.sync_copy(w1_hbm.at[pl.ds(base * E, 64 * E)], wv1)
    for h in range(2):
        b2 = base + h * 32
        pltpu.sync_copy(pos0_hbm.at[pl.ds(b2, 32)], p0v)
        pltpu.sync_copy(pos1_hbm.at[pl.ds(b2, 32)], p1v)
        pltpu.async_copy(y_hbm.at[p0v], buf0, sem).wait()
        pltpu.async_copy(y_hbm.at[p1v], buf1, sem).wait()

        def body(j, _):
            jj = h * 32 + j
            a = wv0[pl.ds(jj * E, E)]
            bwt = wv1[pl.ds(jj * E, E)]
            for c in range(D // E):
                s = c * E
                v = (buf0[j, pl.ds(s, E)] * a + buf1[j, pl.ds(s, E)] * bwt)
                buf0[j, pl.ds(s, E)] = v
            return 0

        lax.fori_loop(0, 32, body, 0)
        pltpu.sync_copy(buf0, out_hbm.at[pl.ds(b2, 32)])


def _combine(y, pos0, pos1, w0f, w1f):
    mesh = plsc.VectorSubcoreMesh(core_axis_name="c", subcore_axis_name="s")
    kfn = functools.partial(
        pl.kernel, mesh=mesh,
        out_type=jax.ShapeDtypeStruct((L_TOK, D), jnp.float32),
        scratch_types=[
            pltpu.VMEM((32,), jnp.int32),
            pltpu.VMEM((32,), jnp.int32),
            pltpu.VMEM((32, D), jnp.float32),
            pltpu.VMEM((32, D), jnp.float32),
            pltpu.VMEM((64 * E,), jnp.float32),
            pltpu.VMEM((64 * E,), jnp.float32),
            pltpu.SemaphoreType.DMA,
        ],
    )(_combine_body)
    return kfn(y, pos0, pos1, w0f, w1f)


_TOK_IDS = np.arange(2 * L_TOK, dtype=np.int32) // 2


def kernel(x, gate_W, w1, w2):
    Bq, Lq, Dq = x.shape
    x_flat = x.reshape(-1, Dq)
    pos0, pos1, w0s, w1s, be, valid = _routing(x_flat, gate_W.T)

    pos0 = pos0.reshape(-1)
    pos1 = pos1.reshape(-1)
    posd = jnp.stack([pos0, pos1], axis=1).reshape(NW * 2, 64)
    tok = jnp.asarray(_TOK_IDS).reshape(NW * 2, 64)

    xs = _dispatch(x_flat, tok, posd)
    y = _ffn(be.reshape(-1), valid.reshape(-1), xs, w1, w2)
    out = _combine(y, pos0, pos1, w0s.reshape(-1), w1s.reshape(-1))
    return out.reshape(Bq, Lq, Dq)
